# baseline (speedup 1.0000x reference)
import functools

import jax
import jax.numpy as jnp
from jax import lax
from jax.experimental import pallas as pl
from jax.experimental.pallas import tpu as pltpu
from jax.experimental.pallas import tpu_sc as plsc

LATENT = 64
PAIR = 2 * LATENT
GCHUNK = 128


@functools.cache
def _build(B: int):
    info = plsc.get_sparse_core_info()
    NC, NS, L = info.num_cores, info.num_subcores, info.num_lanes
    NW = NC * NS
    bpw = B // NW
    nchunks = bpw // GCHUNK
    gpc = GCHUNK // L
    idx_rows = B // GCHUNK
    rows_per_w = bpw // GCHUNK
    NBUF = 2

    mesh = plsc.VectorSubcoreMesh(core_axis_name="c", subcore_axis_name="s")

    @functools.partial(
        pl.kernel,
        mesh=mesh,
        out_type=jax.ShapeDtypeStruct((B,), jnp.float32),
        scratch_types=[
            pltpu.VMEM((rows_per_w, GCHUNK), jnp.int32),
            pltpu.VMEM((rows_per_w, GCHUNK), jnp.int32),
            pltpu.VMEM((rows_per_w, GCHUNK), jnp.int32),
            pltpu.VMEM((rows_per_w, GCHUNK), jnp.int32),
            pltpu.VMEM((NBUF, GCHUNK, PAIR), jnp.float32),
            pltpu.VMEM((NBUF, GCHUNK, PAIR), jnp.float32),
            pltpu.VMEM((LATENT,), jnp.float32),
            pltpu.VMEM((bpw,), jnp.float32),
            pltpu.SemaphoreType.DMA,
            pltpu.SemaphoreType.DMA,
        ],
        compiler_params=pltpu.CompilerParams(needs_layout_passes=False),
    )
    def gmf(upair_hbm, ipair_hbm, ucol_hbm, icol_hbm, utab_hbm, itab_hbm,
            w_hbm, out_hbm,
            upair_v, ipair_v, ucol_v, icol_v, urows_v, irows_v, w_v, out_v,
            sem_u, sem_i):
        wid = lax.axis_index("s") * NC + lax.axis_index("c")
        base = wid * bpw
        row0 = wid * rows_per_w

        pltpu.sync_copy(upair_hbm.at[pl.ds(row0, rows_per_w)], upair_v)
        pltpu.sync_copy(ipair_hbm.at[pl.ds(row0, rows_per_w)], ipair_v)
        pltpu.sync_copy(ucol_hbm.at[pl.ds(row0, rows_per_w)], ucol_v)
        pltpu.sync_copy(icol_hbm.at[pl.ds(row0, rows_per_w)], icol_v)
        pltpu.sync_copy(w_hbm, w_v)

        def fire(c):
            slot = c % NBUF
            return (
                pltpu.async_copy(utab_hbm.at[upair_v.at[c]], urows_v.at[slot], sem_u),
                pltpu.async_copy(itab_hbm.at[ipair_v.at[c]], irows_v.at[slot], sem_i),
            )

        handles = {}
        for c in range(min(NBUF, nchunks)):
            handles[c] = fire(c)

        lanes = lax.iota(jnp.int32, L)
        w_chunks = [w_v[pl.ds(k * L, L)] for k in range(LATENT // L)]

        for c in range(nchunks):
            slot = c % NBUF
            for h in handles.pop(c):
                h.wait()

            def group_body(g, _, c=c, slot=slot):
                row_idx = g * L + lanes
                ucol = ucol_v[c, pl.ds(g * L, L)]
                icol = icol_v[c, pl.ds(g * L, L)]
                acc = jnp.zeros((L,), jnp.float32)
                for d in range(LATENT):
                    u = plsc.load_gather(
                        urows_v, [jnp.full((L,), slot, jnp.int32), row_idx, ucol + d]
                    )
                    v = plsc.load_gather(
                        irows_v, [jnp.full((L,), slot, jnp.int32), row_idx, icol + d]
                    )
                    acc = acc + (u * v) * w_chunks[d // L][d % L]
                p = 1.0 / (1.0 + jnp.exp(-acc))
                out_v[pl.ds(c * GCHUNK + g * L, L)] = p
                return _

            lax.fori_loop(0, gpc, group_body, None)

            if c + NBUF < nchunks:
                handles[c + NBUF] = fire(c + NBUF)

        pltpu.sync_copy(out_v, out_hbm.at[pl.ds(base, bpw)])

    return gmf, idx_rows


def kernel(user_indices, item_indices, user_weight, item_weight, linear_weight):
    B = user_indices.shape[0]
    gmf, idx_rows = _build(B)
    uidx = user_indices.astype(jnp.int32)
    iidx = item_indices.astype(jnp.int32)
    upair = (uidx >> 1).reshape(idx_rows, GCHUNK)
    ipair = (iidx >> 1).reshape(idx_rows, GCHUNK)
    ucol = ((uidx & 1) * LATENT).reshape(idx_rows, GCHUNK)
    icol = ((iidx & 1) * LATENT).reshape(idx_rows, GCHUNK)
    utab = user_weight.reshape(user_weight.shape[0] // 2, PAIR)
    itab = item_weight.reshape(item_weight.shape[0] // 2, PAIR)
    w = linear_weight.reshape(LATENT).astype(jnp.float32)
    out = gmf(upair, ipair, ucol, icol, utab, itab, w)
    return out.reshape(B, 1)


# speedup vs baseline: 1.0171x; 1.0171x over previous
import functools

import jax
import jax.numpy as jnp
from jax import lax
from jax.experimental import pallas as pl
from jax.experimental.pallas import tpu as pltpu
from jax.experimental.pallas import tpu_sc as plsc

LATENT = 64
PAIR = 2 * LATENT
GCHUNK = 128


@functools.cache
def _build_gather(B: int):
    info = plsc.get_sparse_core_info()
    NC, NS, L = info.num_cores, info.num_subcores, info.num_lanes
    NW = NC * NS
    bpw = B // NW
    nchunks = bpw // GCHUNK
    gpc = GCHUNK // L
    rows_per_w = bpw // GCHUNK
    NBUF = 2

    mesh = plsc.VectorSubcoreMesh(core_axis_name="c", subcore_axis_name="s")

    @functools.partial(
        pl.kernel,
        mesh=mesh,
        out_type=jax.ShapeDtypeStruct((LATENT, B), jnp.float32),
        scratch_types=[
            pltpu.VMEM((rows_per_w, GCHUNK), jnp.int32),
            pltpu.VMEM((rows_per_w, GCHUNK), jnp.int32),
            pltpu.VMEM((NBUF, GCHUNK, PAIR), jnp.float32),
            pltpu.VMEM((LATENT, bpw), jnp.float32),
            pltpu.SemaphoreType.DMA,
        ],
        compiler_params=pltpu.CompilerParams(needs_layout_passes=False),
    )
    def gather_t(pidx_hbm, cofs_hbm, tab_hbm, out_hbm,
                 pidx_v, cofs_v, rows_v, gout_v, sem):
        wid = lax.axis_index("s") * NC + lax.axis_index("c")
        base = wid * bpw
        row0 = wid * rows_per_w

        pltpu.sync_copy(pidx_hbm.at[pl.ds(row0, rows_per_w)], pidx_v)
        pltpu.sync_copy(cofs_hbm.at[pl.ds(row0, rows_per_w)], cofs_v)

        def fire(c):
            return pltpu.async_copy(
                tab_hbm.at[pidx_v.at[c]], rows_v.at[c % NBUF], sem
            )

        handles = {c: fire(c) for c in range(min(NBUF, nchunks))}

        lanes = lax.iota(jnp.int32, L)

        for c in range(nchunks):
            slot = c % NBUF
            handles.pop(c).wait()

            def group_body(g, _, c=c, slot=slot):
                row_idx = g * L + lanes
                co = cofs_v[c, pl.ds(g * L, L)]
                sl = jnp.full((L,), slot, jnp.int32)
                for d in range(LATENT):
                    v = plsc.load_gather(rows_v, [sl, row_idx, co + d])
                    gout_v[d, pl.ds(c * GCHUNK + g * L, L)] = v
                return _

            lax.fori_loop(0, gpc, group_body, None)

            if c + NBUF < nchunks:
                handles[c + NBUF] = fire(c + NBUF)

        pltpu.sync_copy(gout_v, out_hbm.at[:, pl.ds(base, bpw)])

    return gather_t


@functools.cache
def _build_combine(B: int):
    def body(gu_ref, gi_ref, w_ref, out_ref):
        prod = gu_ref[...] * gi_ref[...]
        s = jnp.sum(prod * w_ref[...], axis=0, keepdims=True)
        out_ref[...] = 1.0 / (1.0 + jnp.exp(-s))

    return pl.pallas_call(
        body,
        out_shape=jax.ShapeDtypeStruct((1, B), jnp.float32),
    )


def kernel(user_indices, item_indices, user_weight, item_weight, linear_weight):
    B = user_indices.shape[0]
    idx_rows = B // GCHUNK
    gather_t = _build_gather(B)
    combine = _build_combine(B)

    uidx = user_indices.astype(jnp.int32)
    iidx = item_indices.astype(jnp.int32)
    upair = (uidx >> 1).reshape(idx_rows, GCHUNK)
    ipair = (iidx >> 1).reshape(idx_rows, GCHUNK)
    ucol = ((uidx & 1) * LATENT).reshape(idx_rows, GCHUNK)
    icol = ((iidx & 1) * LATENT).reshape(idx_rows, GCHUNK)
    utab = user_weight.reshape(user_weight.shape[0] // 2, PAIR)
    itab = item_weight.reshape(item_weight.shape[0] // 2, PAIR)
    w = linear_weight.reshape(LATENT, 1).astype(jnp.float32)

    gu = gather_t(upair, ucol, utab)
    gi = gather_t(ipair, icol, itab)
    out = combine(gu, gi, w)
    return out.reshape(B, 1)


# speedup vs baseline: 1.5606x; 1.5343x over previous
import functools

import jax
import jax.numpy as jnp
from jax import lax
from jax.experimental import pallas as pl
from jax.experimental.pallas import tpu as pltpu
from jax.experimental.pallas import tpu_sc as plsc

LATENT = 64
CHUNK = 32
NBUF = 2


@functools.cache
def _build_gather(B: int):
    info = plsc.get_sparse_core_info()
    NC, NS, L = info.num_cores, info.num_subcores, info.num_lanes
    NW = NC * NS
    bpw = B // NW
    nchunks = bpw // CHUNK
    gpc = CHUNK // L

    mesh = plsc.VectorSubcoreMesh(core_axis_name="c", subcore_axis_name="s")

    @functools.partial(
        pl.kernel,
        mesh=mesh,
        out_type=jax.ShapeDtypeStruct((LATENT, B), jnp.float32),
        scratch_types=[
            pltpu.VMEM((bpw,), jnp.int32),
            pltpu.VMEM((NBUF, CHUNK, 8, LATENT), jnp.float32),
            pltpu.VMEM((LATENT, bpw), jnp.float32),
            pltpu.SemaphoreType.DMA,
            pltpu.SemaphoreType.DMA,
        ],
        compiler_params=pltpu.CompilerParams(needs_layout_passes=False),
    )
    def gather_t(idx_hbm, tab_hbm, dummy_hbm, out_hbm,
                 idx_v, blk_v, gout_v, sem0, sem1):
        wid = lax.axis_index("s") * NC + lax.axis_index("c")
        base = wid * bpw

        pltpu.sync_copy(idx_hbm.at[pl.ds(base, bpw)], idx_v)

        lanes = lax.iota(jnp.int32, L)
        sems = (sem0, sem1)

        def fire(c, slot):
            for g in range(gpc):
                rv = idx_v[pl.ds(c * CHUNK + g * L, L)]
                for j in range(L):
                    r = rv[j]
                    r0 = pl.multiple_of((r >> 3) << 3, 8)
                    pltpu.async_copy(
                        tab_hbm.at[pl.ds(r0, 8), :],
                        blk_v.at[slot, g * L + j],
                        sems[slot],
                    )

        def drain(slot):
            pltpu.make_async_copy(dummy_hbm, blk_v.at[slot], sems[slot]).wait()

        def compute(c, slot):
            for g in range(gpc):
                e0 = c * CHUNK + g * L
                em = idx_v[pl.ds(e0, L)] & 7
                eid = lanes + g * L
                sl = jnp.full((L,), slot, jnp.int32)
                for d in range(LATENT):
                    v = plsc.load_gather(
                        blk_v, [sl, eid, em, jnp.full((L,), d, jnp.int32)]
                    )
                    gout_v[d, pl.ds(e0, L)] = v

        fire(0, 0)

        def body(k, _):
            c0 = 2 * k
            fire(c0 + 1, 1)
            drain(0)
            compute(c0, 0)

            @pl.when(k < nchunks // 2 - 1)
            def _fire_next():
                fire(c0 + 2, 0)

            drain(1)
            compute(c0 + 1, 1)
            return _

        lax.fori_loop(0, nchunks // 2, body, None)

        pltpu.sync_copy(gout_v, out_hbm.at[:, pl.ds(base, bpw)])

    return gather_t


@functools.cache
def _build_combine(B: int):
    def body(gu_ref, gi_ref, w_ref, out_ref):
        prod = gu_ref[...] * gi_ref[...]
        s = jnp.sum(prod * w_ref[...], axis=0, keepdims=True)
        out_ref[...] = 1.0 / (1.0 + jnp.exp(-s))

    return pl.pallas_call(
        body,
        out_shape=jax.ShapeDtypeStruct((1, B), jnp.float32),
    )


def kernel(user_indices, item_indices, user_weight, item_weight, linear_weight):
    B = user_indices.shape[0]
    gather_t = _build_gather(B)
    combine = _build_combine(B)

    uidx = user_indices.astype(jnp.int32)
    iidx = item_indices.astype(jnp.int32)
    w = linear_weight.reshape(LATENT, 1).astype(jnp.float32)
    dummy = jnp.zeros((CHUNK, 8, LATENT), jnp.float32)

    gu = gather_t(uidx, user_weight, dummy)
    gi = gather_t(iidx, item_weight, dummy)
    out = combine(gu, gi, w)
    return out.reshape(B, 1)


# speedup vs baseline: 2.3641x; 1.5149x over previous
import functools

import jax
import jax.numpy as jnp
from jax import lax
from jax.experimental import pallas as pl
from jax.experimental.pallas import tpu as pltpu
from jax.experimental.pallas import tpu_sc as plsc

LATENT = 64
SLAB = 128
CHUNK = 4
NBUF = 2


@functools.cache
def _build_gather(B: int):
    info = plsc.get_sparse_core_info()
    NC, NS, L = info.num_cores, info.num_subcores, info.num_lanes
    NW = NC * NS
    bpw = B // NW
    nchunks = bpw // CHUNK

    mesh = plsc.VectorSubcoreMesh(core_axis_name="c", subcore_axis_name="s")

    @functools.partial(
        pl.kernel,
        mesh=mesh,
        out_type=jax.ShapeDtypeStruct((LATENT, B), jnp.float32),
        scratch_types=[
            pltpu.VMEM((bpw,), jnp.int32),
            pltpu.VMEM((NBUF, CHUNK, LATENT, SLAB), jnp.float32),
            pltpu.VMEM((LATENT, bpw), jnp.float32),
            pltpu.SemaphoreType.DMA,
            pltpu.SemaphoreType.DMA,
        ],
        compiler_params=pltpu.CompilerParams(needs_layout_passes=False),
    )
    def gather_t(idx_hbm, tab_hbm, dummy_hbm, out_hbm,
                 idx_v, slab_v, gout_v, sem0, sem1):
        wid = lax.axis_index("s") * NC + lax.axis_index("c")
        base = wid * bpw

        pltpu.sync_copy(idx_hbm.at[pl.ds(base, bpw)], idx_v)

        lanes = lax.iota(jnp.int32, L)
        sems = (sem0, sem1)
        dvecs = [lanes + k * L for k in range(LATENT // L)]

        def rvec(c):
            return idx_v[pl.ds(c * CHUNK, L)]

        def fire(c, slot):
            rv = rvec(c)
            for j in range(CHUNK):
                r = rv[j]
                c0 = pl.multiple_of((r >> 7) << 7, SLAB)
                pltpu.async_copy(
                    tab_hbm.at[:, pl.ds(c0, SLAB)],
                    slab_v.at[slot, j],
                    sems[slot],
                )

        def drain(slot):
            pltpu.make_async_copy(dummy_hbm, slab_v.at[slot], sems[slot]).wait()

        def compute(c, slot):
            rv = rvec(c)
            for j in range(CHUNK):
                col = jnp.full((L,), rv[j] & (SLAB - 1), jnp.int32)
                epos = jnp.full((L,), c * CHUNK + j, jnp.int32)
                sl = jnp.full((L,), slot, jnp.int32)
                ej = jnp.full((L,), j, jnp.int32)
                for k in range(LATENT // L):
                    v = plsc.load_gather(slab_v, [sl, ej, dvecs[k], col])
                    plsc.store_scatter(gout_v, [dvecs[k], epos], v)

        fire(0, 0)

        def body(k, _):
            c0 = 2 * k
            fire(c0 + 1, 1)
            drain(0)
            compute(c0, 0)

            @pl.when(k < nchunks // 2 - 1)
            def _fire_next():
                fire(c0 + 2, 0)

            drain(1)
            compute(c0 + 1, 1)
            return _

        lax.fori_loop(0, nchunks // 2, body, None)

        pltpu.sync_copy(gout_v, out_hbm.at[:, pl.ds(base, bpw)])

    return gather_t


@functools.cache
def _build_combine(B: int):
    def body(gu_ref, gi_ref, w_ref, out_ref):
        prod = gu_ref[...] * gi_ref[...]
        s = jnp.sum(prod * w_ref[...], axis=0, keepdims=True)
        out_ref[...] = 1.0 / (1.0 + jnp.exp(-s))

    return pl.pallas_call(
        body,
        out_shape=jax.ShapeDtypeStruct((1, B), jnp.float32),
    )


def kernel(user_indices, item_indices, user_weight, item_weight, linear_weight):
    B = user_indices.shape[0]
    gather_t = _build_gather(B)
    combine = _build_combine(B)

    uidx = user_indices.astype(jnp.int32)
    iidx = item_indices.astype(jnp.int32)
    w = linear_weight.reshape(LATENT, 1).astype(jnp.float32)
    dummy = jnp.zeros((CHUNK, LATENT, SLAB), jnp.float32)

    gu = gather_t(uidx, user_weight.T, dummy)
    gi = gather_t(iidx, item_weight.T, dummy)
    out = combine(gu, gi, w)
    return out.reshape(B, 1)


# speedup vs baseline: 2.6041x; 1.1015x over previous
import functools

import jax
import jax.numpy as jnp
from jax import lax
from jax.experimental import pallas as pl
from jax.experimental.pallas import tpu as pltpu
from jax.experimental.pallas import tpu_sc as plsc

LATENT = 64
SLAB = 128
CHUNK = 4
NBUF = 3
FLUSH = 128


@functools.cache
def _build_gather(B: int):
    info = plsc.get_sparse_core_info()
    NC, NS, L = info.num_cores, info.num_subcores, info.num_lanes
    NW = NC * NS
    bpw = B // NW
    nchunks = bpw // CHUNK
    fchunks = FLUSH // CHUNK

    mesh = plsc.VectorSubcoreMesh(core_axis_name="c", subcore_axis_name="s")

    @functools.partial(
        pl.kernel,
        mesh=mesh,
        out_type=jax.ShapeDtypeStruct((LATENT, B), jnp.float32),
        scratch_types=[
            pltpu.VMEM((bpw + 16,), jnp.int32),
            pltpu.VMEM((NBUF, CHUNK, LATENT, SLAB), jnp.float32),
            pltpu.VMEM((LATENT, FLUSH), jnp.float32),
            pltpu.SemaphoreType.DMA,
            pltpu.SemaphoreType.DMA,
            pltpu.SemaphoreType.DMA,
        ],
        compiler_params=pltpu.CompilerParams(needs_layout_passes=False),
    )
    def gather_t(idx_hbm, tab_hbm, dummy_hbm, out_hbm,
                 idx_v, slab_v, gst_v, sem0, sem1, sem2):
        wid = lax.axis_index("s") * NC + lax.axis_index("c")
        base = wid * bpw

        pltpu.sync_copy(idx_hbm.at[pl.ds(base, bpw)], idx_v.at[pl.ds(0, bpw)])

        lanes = lax.iota(jnp.int32, L)
        sems = (sem0, sem1, sem2)
        dvecs = [lanes + k * L for k in range(LATENT // L)]

        def rvec(c):
            return idx_v[pl.ds(c * CHUNK, L)]

        def fire(c, slot):
            rv = rvec(c)
            for j in range(CHUNK):
                r = rv[j]
                c0 = pl.multiple_of((r >> 7) << 7, SLAB)
                pltpu.async_copy(
                    tab_hbm.at[:, pl.ds(c0, SLAB)],
                    slab_v.at[slot, j],
                    sems[slot],
                )

        def fire_if(c, slot):
            @pl.when(c < nchunks)
            def _():
                fire(c, slot)

        def drain(slot):
            pltpu.make_async_copy(dummy_hbm, slab_v.at[slot], sems[slot]).wait()

        def step(c, slot):
            drain(slot)
            rv = rvec(c)
            for j in range(CHUNK):
                col = jnp.full((L,), rv[j] & (SLAB - 1), jnp.int32)
                epos = jnp.full((L,), (c * CHUNK + j) & (FLUSH - 1), jnp.int32)
                sl = jnp.full((L,), slot, jnp.int32)
                ej = jnp.full((L,), j, jnp.int32)
                for k in range(LATENT // L):
                    v = plsc.load_gather(slab_v, [sl, ej, dvecs[k], col])
                    plsc.store_scatter(gst_v, [dvecs[k], epos], v)

            @pl.when((c + 1) % fchunks == 0)
            def _flush():
                blk = c // fchunks
                pltpu.sync_copy(
                    gst_v, out_hbm.at[:, pl.ds(base + blk * FLUSH, FLUSH)]
                )

        fire(0, 0)
        fire(1, 1)
        fire(2, 2)

        def body(k, _):
            c = 3 * k
            step(c, 0)
            fire_if(c + 3, 0)
            step(c + 1, 1)
            fire_if(c + 4, 1)
            step(c + 2, 2)
            fire_if(c + 5, 2)
            return _

        lax.fori_loop(0, nchunks // 3, body, None)

        step(nchunks - 2, 0)
        step(nchunks - 1, 1)

    return gather_t


@functools.cache
def _build_combine(B: int):
    def body(gu_ref, gi_ref, w_ref, out_ref):
        prod = gu_ref[...] * gi_ref[...]
        s = jnp.sum(prod * w_ref[...], axis=0, keepdims=True)
        out_ref[...] = 1.0 / (1.0 + jnp.exp(-s))

    return pl.pallas_call(
        body,
        out_shape=jax.ShapeDtypeStruct((1, B), jnp.float32),
    )


def kernel(user_indices, item_indices, user_weight, item_weight, linear_weight):
    B = user_indices.shape[0]
    gather_t = _build_gather(B)
    combine = _build_combine(B)

    uidx = user_indices.astype(jnp.int32)
    iidx = item_indices.astype(jnp.int32)
    w = linear_weight.reshape(LATENT, 1).astype(jnp.float32)
    dummy = jnp.zeros((CHUNK, LATENT, SLAB), jnp.float32)

    gu = gather_t(uidx, user_weight.T, dummy)
    gi = gather_t(iidx, item_weight.T, dummy)
    out = combine(gu, gi, w)
    return out.reshape(B, 1)


# speedup vs baseline: 2.6740x; 1.0268x over previous
import functools

import jax
import jax.numpy as jnp
from jax import lax
from jax.experimental import pallas as pl
from jax.experimental.pallas import tpu as pltpu
from jax.experimental.pallas import tpu_sc as plsc

LATENT = 64
SLAB = 128
CHUNK = 2
NBUF = 3


@functools.cache
def _build(B: int):
    info = plsc.get_sparse_core_info()
    NC, NS, L = info.num_cores, info.num_subcores, info.num_lanes
    NW = NC * NS
    bpw = B // NW
    nchunks = bpw // CHUNK

    mesh = plsc.VectorSubcoreMesh(core_axis_name="c", subcore_axis_name="s")

    @functools.partial(
        pl.kernel,
        mesh=mesh,
        out_type=jax.ShapeDtypeStruct((B,), jnp.float32),
        scratch_types=[
            pltpu.VMEM((bpw + 16,), jnp.int32),
            pltpu.VMEM((bpw + 16,), jnp.int32),
            pltpu.VMEM((NBUF, 2, CHUNK, LATENT, SLAB), jnp.float32),
            pltpu.VMEM((LATENT,), jnp.float32),
            pltpu.VMEM((bpw,), jnp.float32),
            pltpu.SemaphoreType.DMA,
            pltpu.SemaphoreType.DMA,
            pltpu.SemaphoreType.DMA,
        ],
        compiler_params=pltpu.CompilerParams(needs_layout_passes=False),
    )
    def gmf(uidx_hbm, iidx_hbm, utab_hbm, itab_hbm, w_hbm, dummy_hbm, out_hbm,
            uidx_v, iidx_v, slab_v, w_v, out_v, sem0, sem1, sem2):
        wid = lax.axis_index("s") * NC + lax.axis_index("c")
        base = wid * bpw

        pltpu.sync_copy(uidx_hbm.at[pl.ds(base, bpw)], uidx_v.at[pl.ds(0, bpw)])
        pltpu.sync_copy(iidx_hbm.at[pl.ds(base, bpw)], iidx_v.at[pl.ds(0, bpw)])
        pltpu.sync_copy(w_hbm, w_v)

        lanes = lax.iota(jnp.int32, L)
        sems = (sem0, sem1, sem2)
        dvecs = [lanes + k * L for k in range(LATENT // L)]
        w_chunks = [w_v[pl.ds(k * L, L)] for k in range(LATENT // L)]

        def fire(c, slot):
            ruv = uidx_v[pl.ds(c * CHUNK, L)]
            riv = iidx_v[pl.ds(c * CHUNK, L)]
            for j in range(CHUNK):
                r = ruv[j]
                c0 = pl.multiple_of((r >> 7) << 7, SLAB)
                pltpu.async_copy(
                    utab_hbm.at[:, pl.ds(c0, SLAB)],
                    slab_v.at[slot, 0, j],
                    sems[slot],
                )
                s = riv[j]
                c1 = pl.multiple_of((s >> 7) << 7, SLAB)
                pltpu.async_copy(
                    itab_hbm.at[:, pl.ds(c1, SLAB)],
                    slab_v.at[slot, 1, j],
                    sems[slot],
                )

        def fire_if(c, slot):
            @pl.when(c < nchunks)
            def _():
                fire(c, slot)

        def step(c, slot):
            pltpu.make_async_copy(dummy_hbm, slab_v.at[slot], sems[slot]).wait()
            ruv = uidx_v[pl.ds(c * CHUNK, L)]
            riv = iidx_v[pl.ds(c * CHUNK, L)]
            for j in range(CHUNK):
                ucol = jnp.full((L,), ruv[j] & (SLAB - 1), jnp.int32)
                icol = jnp.full((L,), riv[j] & (SLAB - 1), jnp.int32)
                sl = jnp.full((L,), slot, jnp.int32)
                ej = jnp.full((L,), j, jnp.int32)
                zero = jnp.zeros((L,), jnp.int32)
                one = jnp.full((L,), 1, jnp.int32)
                acc = jnp.zeros((L,), jnp.float32)
                for k in range(LATENT // L):
                    u = plsc.load_gather(slab_v, [sl, zero, ej, dvecs[k], ucol])
                    v = plsc.load_gather(slab_v, [sl, one, ej, dvecs[k], icol])
                    acc = acc + (u * v) * w_chunks[k]
                logit = jnp.sum(acc)
                epos = jnp.full((L,), c * CHUNK + j, jnp.int32)
                plsc.store_scatter(
                    out_v, [epos], jnp.broadcast_to(logit, (L,)),
                    mask=lanes == 0,
                )

        fire(0, 0)
        fire(1, 1)
        fire(2, 2)

        def body(k, _):
            c = 3 * k
            step(c, 0)
            fire_if(c + 3, 0)
            step(c + 1, 1)
            fire_if(c + 4, 1)
            step(c + 2, 2)
            fire_if(c + 5, 2)
            return _

        lax.fori_loop(0, nchunks // 3, body, None)
        step(nchunks - 1, 0)

        for t in range(bpw // L):
            x = out_v[pl.ds(t * L, L)]
            out_v[pl.ds(t * L, L)] = 1.0 / (1.0 + jnp.exp(-x))

        pltpu.sync_copy(out_v, out_hbm.at[pl.ds(base, bpw)])

    return gmf


def kernel(user_indices, item_indices, user_weight, item_weight, linear_weight):
    B = user_indices.shape[0]
    gmf = _build(B)
    uidx = user_indices.astype(jnp.int32)
    iidx = item_indices.astype(jnp.int32)
    w = linear_weight.reshape(LATENT).astype(jnp.float32)
    dummy = jnp.zeros((2, CHUNK, LATENT, SLAB), jnp.float32)
    out = gmf(uidx, iidx, user_weight.T, item_weight.T, w, dummy)
    return out.reshape(B, 1)
